# baseline (speedup 1.0000x reference)
import jax
import jax.numpy as jnp
from jax import lax
from jax.experimental import pallas as pl
from jax.experimental.pallas import tpu as pltpu
from jax.experimental.pallas import tpu_sc as plsc

NC = 2
NS = 16
NW = NC * NS
L = 16

ROWS, COLS, D = 16384, 200, 5
PER_W = ROWS // NW
IB = 128
JB = 40
NJB = COLS // JB
NSTEP = (PER_W // IB) * NJB


def _body(xt_hbm, w_hbm, out_hbm, x_t0, x_t1, w_t, o_t0, o_t1,
          sx0, sx1, so0, so1):
    wid = lax.axis_index("s") * NC + lax.axis_index("c")
    i0w = wid * PER_W
    pltpu.sync_copy(w_hbm, w_t)
    cvecs = [jnp.full((L,), c, jnp.int32) for c in range(D)]
    x_bufs = (x_t0, x_t1)
    o_bufs = (o_t0, o_t1)
    sx = (sx0, sx1)
    so = (so0, so1)

    def x_cp(step, p):
        ib = step // NJB
        jb = step % NJB
        return pltpu.make_async_copy(
            xt_hbm.at[pl.ds(jb * JB, JB), pl.ds(i0w + ib * IB, IB)],
            x_bufs[p], sx[p])

    def o_cp(step, c, p):
        ib = step // NJB
        jb = step % NJB
        return pltpu.make_async_copy(
            o_bufs[p].at[c],
            out_hbm.at[pl.ds(c * COLS + jb * JB, JB),
                       pl.ds(i0w + ib * IB, IB)],
            so[p])

    x_cp(0, 0).start()
    x_cp(1, 1).start()

    def pair(s, _):
        for p in range(2):
            step = s * 2 + p
            x_t = x_bufs[p]
            o_t = o_bufs[p]
            x_cp(step, p).wait()

            @pl.when(step >= 2)
            def _drain():
                for c in range(D):
                    o_cp(step, c, p).wait()

            @plsc.parallel_loop(0, JB, unroll=1)
            def _grp(jrel):
                @plsc.parallel_loop(0, IB // L, unroll=1)
                def _g(g):
                    xv = x_t[jrel, pl.ds(g * L, L)]
                    for c in range(D):
                        vals = plsc.load_gather(w_t, [cvecs[c], xv])
                        o_t[c, jrel, pl.ds(g * L, L)] = vals

            for c in range(D):
                o_cp(step, c, p).start()

            @pl.when(step + 2 < NSTEP)
            def _next_x():
                x_cp(step + 2, p).start()
        return 0

    lax.fori_loop(0, NSTEP // 2, pair, 0, unroll=False)
    for c in range(D):
        o_cp(NSTEP - 2, c, 0).wait()
        o_cp(NSTEP - 1, c, 1).wait()


@jax.jit
def kernel(x, weight):
    xt = x.astype(jnp.int32).T
    wt = weight.astype(jnp.float32).T
    mesh = plsc.VectorSubcoreMesh(core_axis_name="c", subcore_axis_name="s")
    run = pl.kernel(
        _body,
        out_type=jax.ShapeDtypeStruct((D * COLS, ROWS), jnp.float32),
        mesh=mesh,
        scratch_types=[
            pltpu.VMEM((JB, IB), jnp.int32),
            pltpu.VMEM((JB, IB), jnp.int32),
            pltpu.VMEM((D, 14), jnp.float32),
            pltpu.VMEM((D, JB, IB), jnp.float32),
            pltpu.VMEM((D, JB, IB), jnp.float32),
            pltpu.SemaphoreType.DMA,
            pltpu.SemaphoreType.DMA,
            pltpu.SemaphoreType.DMA,
            pltpu.SemaphoreType.DMA,
        ],
        compiler_params=pltpu.CompilerParams(
            needs_layout_passes=False, use_tc_tiling_on_sc=True),
    )
    out = run(xt, wt)
    return out.reshape(D, COLS, ROWS).transpose(2, 1, 0)


# speedup vs baseline: 1.1293x; 1.1293x over previous
import jax
import jax.numpy as jnp
from jax import lax
from jax.experimental import pallas as pl
from jax.experimental.pallas import tpu as pltpu
from jax.experimental.pallas import tpu_sc as plsc

NC = 2
NS = 16
NW = NC * NS
L = 16

ROWS, COLS, D = 16384, 200, 5
PER_W = ROWS // NW
IB = 256
JB = 40
NJB = COLS // JB
NSTEP = (PER_W // IB) * NJB


def _body(xt_hbm, w_hbm, out_hbm, x_t0, x_t1, w_t, o_t0, o_t1,
          sx0, sx1, so0, so1):
    wid = lax.axis_index("s") * NC + lax.axis_index("c")
    i0w = wid * PER_W
    pltpu.sync_copy(w_hbm, w_t)
    cvecs = [jnp.full((L,), c, jnp.int32) for c in range(D)]
    x_bufs = (x_t0, x_t1)
    o_bufs = (o_t0, o_t1)
    sx = (sx0, sx1)
    so = (so0, so1)

    def x_cp(step, p):
        ib = step // NJB
        jb = step % NJB
        return pltpu.make_async_copy(
            xt_hbm.at[pl.ds(jb * JB, JB), pl.ds(i0w + ib * IB, IB)],
            x_bufs[p], sx[p])

    def o_cp(step, c, p):
        ib = step // NJB
        jb = step % NJB
        return pltpu.make_async_copy(
            o_bufs[p].at[c],
            out_hbm.at[pl.ds(c * COLS + jb * JB, JB),
                       pl.ds(i0w + ib * IB, IB)],
            so[p])

    x_cp(0, 0).start()
    x_cp(1, 1).start()

    def pair(s, _):
        for p in range(2):
            step = s * 2 + p
            x_t = x_bufs[p]
            o_t = o_bufs[p]
            x_cp(step, p).wait()

            @pl.when(step >= 2)
            def _drain():
                for c in range(D):
                    o_cp(step, c, p).wait()

            @plsc.parallel_loop(0, JB, unroll=1)
            def _grp(jrel):
                for g in range(IB // L):
                    xv = x_t[jrel, pl.ds(g * L, L)]
                    for c in range(D):
                        vals = plsc.load_gather(w_t, [cvecs[c], xv])
                        o_t[c, jrel, pl.ds(g * L, L)] = vals

            for c in range(D):
                o_cp(step, c, p).start()

            @pl.when(step + 2 < NSTEP)
            def _next_x():
                x_cp(step + 2, p).start()
        return 0

    lax.fori_loop(0, NSTEP // 2, pair, 0, unroll=False)
    for c in range(D):
        o_cp(NSTEP - 2, c, 0).wait()
        o_cp(NSTEP - 1, c, 1).wait()


@jax.jit
def kernel(x, weight):
    xt = x.astype(jnp.int32).T
    wt = weight.astype(jnp.float32).T
    mesh = plsc.VectorSubcoreMesh(core_axis_name="c", subcore_axis_name="s")
    run = pl.kernel(
        _body,
        out_type=jax.ShapeDtypeStruct((D * COLS, ROWS), jnp.float32),
        mesh=mesh,
        scratch_types=[
            pltpu.VMEM((JB, IB), jnp.int32),
            pltpu.VMEM((JB, IB), jnp.int32),
            pltpu.VMEM((D, 14), jnp.float32),
            pltpu.VMEM((D, JB, IB), jnp.float32),
            pltpu.VMEM((D, JB, IB), jnp.float32),
            pltpu.SemaphoreType.DMA,
            pltpu.SemaphoreType.DMA,
            pltpu.SemaphoreType.DMA,
            pltpu.SemaphoreType.DMA,
        ],
        compiler_params=pltpu.CompilerParams(
            needs_layout_passes=False, use_tc_tiling_on_sc=True),
    )
    out = run(xt, wt)
    return out.reshape(D, COLS, ROWS).transpose(2, 1, 0)


# speedup vs baseline: 1.2194x; 1.0798x over previous
import jax
import jax.numpy as jnp
from jax import lax
from jax.experimental import pallas as pl
from jax.experimental.pallas import tpu as pltpu
from jax.experimental.pallas import tpu_sc as plsc

NC = 2
NS = 16
NW = NC * NS
L = 16

ROWS, COLS, D = 16384, 200, 5
PER_W = ROWS // NW
IB = 128
JB = 40
NJB = COLS // JB
NSTEP = (PER_W // IB) * NJB


def _body(xt_hbm, w_hbm, out_hbm, x_t0, x_t1, w_t, o_t0, o_t1,
          sx0, sx1, so0, so1):
    wid = lax.axis_index("s") * NC + lax.axis_index("c")
    i0w = wid * PER_W
    pltpu.sync_copy(w_hbm, w_t)
    cvecs = [jnp.full((L,), c, jnp.int32) for c in range(D)]
    x_bufs = (x_t0, x_t1)
    o_bufs = (o_t0, o_t1)
    sx = (sx0, sx1)
    so = (so0, so1)

    def x_cp(step, p):
        ib = step // NJB
        jb = step % NJB
        return pltpu.make_async_copy(
            xt_hbm.at[pl.ds(jb * JB, JB), pl.ds(i0w + ib * IB, IB)],
            x_bufs[p], sx[p])

    def o_cp(step, c, p):
        ib = step // NJB
        jb = step % NJB
        return pltpu.make_async_copy(
            o_bufs[p].at[c],
            out_hbm.at[pl.ds(c * COLS + jb * JB, JB),
                       pl.ds(i0w + ib * IB, IB)],
            so[p])

    x_cp(0, 0).start()
    x_cp(1, 1).start()

    def pair(s, _):
        for p in range(2):
            step = s * 2 + p
            x_t = x_bufs[p]
            o_t = o_bufs[p]
            x_cp(step, p).wait()

            @pl.when(step >= 2)
            def _drain():
                for c in range(D):
                    o_cp(step, c, p).wait()

            @plsc.parallel_loop(0, JB, unroll=1)
            def _grp(jrel):
                for g in range(IB // L):
                    xv = x_t[jrel, pl.ds(g * L, L)]
                    for c in range(D):
                        vals = plsc.load_gather(w_t, [cvecs[c], xv])
                        o_t[c, jrel, pl.ds(g * L, L)] = vals

            for c in range(D):
                o_cp(step, c, p).start()

            @pl.when(step + 2 < NSTEP)
            def _next_x():
                x_cp(step + 2, p).start()
        return 0

    lax.fori_loop(0, NSTEP // 2, pair, 0, unroll=False)
    for c in range(D):
        o_cp(NSTEP - 2, c, 0).wait()
        o_cp(NSTEP - 1, c, 1).wait()


@jax.jit
def kernel(x, weight):
    xt = x.astype(jnp.int32).T
    wt = weight.astype(jnp.float32).T
    mesh = plsc.VectorSubcoreMesh(core_axis_name="c", subcore_axis_name="s")
    run = pl.kernel(
        _body,
        out_type=jax.ShapeDtypeStruct((D * COLS, ROWS), jnp.float32),
        mesh=mesh,
        scratch_types=[
            pltpu.VMEM((JB, IB), jnp.int32),
            pltpu.VMEM((JB, IB), jnp.int32),
            pltpu.VMEM((D, 14), jnp.float32),
            pltpu.VMEM((D, JB, IB), jnp.float32),
            pltpu.VMEM((D, JB, IB), jnp.float32),
            pltpu.SemaphoreType.DMA,
            pltpu.SemaphoreType.DMA,
            pltpu.SemaphoreType.DMA,
            pltpu.SemaphoreType.DMA,
        ],
        compiler_params=pltpu.CompilerParams(
            needs_layout_passes=False, use_tc_tiling_on_sc=True),
    )
    out = run(xt, wt)
    return out.reshape(D, COLS, ROWS).transpose(2, 1, 0)


# speedup vs baseline: 1.2262x; 1.0055x over previous
import jax
import jax.numpy as jnp
from jax import lax
from jax.experimental import pallas as pl
from jax.experimental.pallas import tpu as pltpu
from jax.experimental.pallas import tpu_sc as plsc

NC = 2
NS = 16
NW = NC * NS
L = 16

ROWS, COLS, D = 16384, 200, 5
PER_W = ROWS // NW
IB = 128
JB = 40
NJB = COLS // JB
NSTEP = (PER_W // IB) * NJB


def _body(xt_hbm, w_hbm, out_hbm, x_t0, x_t1, w_t, o_t0, o_t1,
          sx0, sx1, so0, so1):
    wid = lax.axis_index("s") * NC + lax.axis_index("c")
    i0w = wid * PER_W
    cvecs = [jnp.full((L,), c, jnp.int32) for c in range(D)]
    x_bufs = (x_t0, x_t1)
    o_bufs = (o_t0, o_t1)
    sx = (sx0, sx1)
    so = (so0, so1)

    def x_cp(step, p):
        ib = step // NJB
        jb = step % NJB
        return pltpu.make_async_copy(
            xt_hbm.at[pl.ds(jb * JB, JB), pl.ds(i0w + ib * IB, IB)],
            x_bufs[p], sx[p])

    def o_cp(step, c, p):
        ib = step // NJB
        jb = step % NJB
        return pltpu.make_async_copy(
            o_bufs[p].at[c],
            out_hbm.at[pl.ds(c * COLS + jb * JB, JB),
                       pl.ds(i0w + ib * IB, IB)],
            so[p])

    x_cp(0, 0).start()
    x_cp(1, 1).start()
    pltpu.sync_copy(w_hbm, w_t)

    def pair(s, _):
        for p in range(2):
            step = s * 2 + p
            x_t = x_bufs[p]
            o_t = o_bufs[p]
            x_cp(step, p).wait()

            @pl.when(step >= 2)
            def _drain():
                for c in range(D):
                    o_cp(step, c, p).wait()

            @plsc.parallel_loop(0, JB, unroll=1)
            def _grp(jrel):
                for g in range(IB // L):
                    xv = x_t[jrel, pl.ds(g * L, L)]
                    for c in range(D):
                        vals = plsc.load_gather(w_t, [cvecs[c], xv])
                        o_t[c, jrel, pl.ds(g * L, L)] = vals

            for c in range(D):
                o_cp(step, c, p).start()

            @pl.when(step + 2 < NSTEP)
            def _next_x():
                x_cp(step + 2, p).start()
        return 0

    lax.fori_loop(0, NSTEP // 2, pair, 0, unroll=False)
    for c in range(D):
        o_cp(NSTEP - 2, c, 0).wait()
        o_cp(NSTEP - 1, c, 1).wait()


@jax.jit
def kernel(x, weight):
    xt = x.astype(jnp.int32).T
    wt = weight.astype(jnp.float32).T
    mesh = plsc.VectorSubcoreMesh(core_axis_name="c", subcore_axis_name="s")
    run = pl.kernel(
        _body,
        out_type=jax.ShapeDtypeStruct((D * COLS, ROWS), jnp.float32),
        mesh=mesh,
        scratch_types=[
            pltpu.VMEM((JB, IB), jnp.int32),
            pltpu.VMEM((JB, IB), jnp.int32),
            pltpu.VMEM((D, 14), jnp.float32),
            pltpu.VMEM((D, JB, IB), jnp.float32),
            pltpu.VMEM((D, JB, IB), jnp.float32),
            pltpu.SemaphoreType.DMA,
            pltpu.SemaphoreType.DMA,
            pltpu.SemaphoreType.DMA,
            pltpu.SemaphoreType.DMA,
        ],
        compiler_params=pltpu.CompilerParams(
            needs_layout_passes=False, use_tc_tiling_on_sc=True),
    )
    out = run(xt, wt)
    return out.reshape(D, COLS, ROWS).transpose(2, 1, 0)
